# Initial kernel scaffold; baseline (speedup 1.0000x reference)
import jax
import jax.numpy as jnp
from jax.experimental import pallas as pl


def kernel(xyz, sparse_xyz, sparse_flow):
    raise NotImplementedError("write your pallas kernel here")


# speedup vs baseline: 27.7887x; 27.7887x over previous
import functools

import jax
import jax.numpy as jnp
from jax.experimental import pallas as pl

_TILE_N = 512


def _upsample_kernel(xq_ref, sx_ref, sf_ref, out_ref, *, S):
    d2 = (
        (xq_ref[:, 0:1] - sx_ref[0:1, :]) ** 2
        + (xq_ref[:, 1:2] - sx_ref[1:2, :]) ** 2
        + (xq_ref[:, 2:3] - sx_ref[2:3, :]) ** 2
    )

    iota = jax.lax.broadcasted_iota(jnp.int32, d2.shape, 1)
    wsum = jnp.zeros((d2.shape[0], 1), jnp.float32)
    acc = [jnp.zeros((d2.shape[0], 1), jnp.float32) for _ in range(3)]
    d = d2
    for _ in range(3):
        mk = jnp.min(d, axis=1, keepdims=True)
        cand = jnp.where(d == mk, iota, S)
        first = jnp.min(cand, axis=1, keepdims=True)
        onehot = iota == first
        w = 1.0 / jnp.maximum(jnp.sqrt(mk), 1e-10)
        for c in range(3):
            f = jnp.sum(jnp.where(onehot, sf_ref[c : c + 1, :], 0.0), axis=1,
                        keepdims=True)
            acc[c] = acc[c] + w * f
        wsum = wsum + w
        d = jnp.where(onehot, jnp.inf, d)

    out = jnp.concatenate([acc[0], acc[1], acc[2]], axis=1) / wsum
    out_ref[...] = jnp.clip(out, -100.0, 100.0)


def kernel(xyz, sparse_xyz, sparse_flow):
    B, C, N = xyz.shape
    S = sparse_xyz.shape[2]
    nt = N // _TILE_N

    xq = jnp.transpose(xyz, (0, 2, 1)).reshape(B * N, C)
    sx = jnp.transpose(sparse_xyz, (1, 0, 2)).reshape(C, B * S)
    sf = jnp.transpose(sparse_flow, (1, 0, 2)).reshape(C, B * S)

    out = pl.pallas_call(
        functools.partial(_upsample_kernel, S=S),
        grid=(B, nt),
        in_specs=[
            pl.BlockSpec((_TILE_N, C), lambda b, t: (b * nt + t, 0)),
            pl.BlockSpec((C, S), lambda b, t: (0, b)),
            pl.BlockSpec((C, S), lambda b, t: (0, b)),
        ],
        out_specs=pl.BlockSpec((_TILE_N, C), lambda b, t: (b * nt + t, 0)),
        out_shape=jax.ShapeDtypeStruct((B * N, C), jnp.float32),
    )(xq, sx, sf)

    return jnp.transpose(out.reshape(B, N, C), (0, 2, 1))
